# Initial kernel scaffold; baseline (speedup 1.0000x reference)
import jax
import jax.numpy as jnp
from jax.experimental import pallas as pl


def kernel(user_ids, item_ids, item_know, priori, condi_p, condi_n, item_diff_w, item_disc_w, uc_w, uc_b, ic_w, ic_b, c1_w, c1_b, c2_w, c2_b):
    raise NotImplementedError("write your pallas kernel here")


# speedup vs baseline: 4.4982x; 4.4982x over previous
import functools

import jax
import jax.numpy as jnp
from jax import lax
from jax.experimental import pallas as pl
from jax.experimental.pallas import tpu as pltpu
from jax.experimental.pallas import tpu_sc as plsc

_N_KNOW = 128
_N_EDGE = 253


def _sc_gather(user_ids, item_ids, priori, condi_p, condi_n, item_diff_w,
               item_disc_w):
    B = user_ids.shape[0]
    info = plsc.get_sparse_core_info()
    nw = info.num_cores * info.num_subcores
    ch = 128
    b_per_w = B // nw
    nch = b_per_w // ch

    mesh = plsc.VectorSubcoreMesh(core_axis_name="c", subcore_axis_name="s")

    out_type = (
        jax.ShapeDtypeStruct((B, _N_KNOW), jnp.float32),
        jax.ShapeDtypeStruct((B, _N_EDGE), jnp.float32),
        jax.ShapeDtypeStruct((B, _N_EDGE), jnp.float32),
        jax.ShapeDtypeStruct((B, _N_KNOW), jnp.float32),
        jax.ShapeDtypeStruct((B, 1), jnp.float32),
    )

    scratch = [
        pltpu.VMEM((nch, ch), jnp.int32),
        pltpu.VMEM((nch, ch), jnp.int32),
        pltpu.VMEM((ch, _N_KNOW), jnp.float32),
        pltpu.VMEM((ch, _N_KNOW), jnp.float32),
        pltpu.VMEM((ch, _N_EDGE), jnp.float32),
        pltpu.VMEM((ch, _N_EDGE), jnp.float32),
        pltpu.VMEM((ch, 1), jnp.float32),
        pltpu.SemaphoreType.DMA,
        pltpu.SemaphoreType.DMA,
    ]

    @functools.partial(pl.kernel, mesh=mesh, out_type=out_type,
                       scratch_types=scratch,
                       compiler_params=pltpu.CompilerParams(
                           use_tc_tiling_on_sc=False))
    def gather_kernel(uid_hbm, iid_hbm, pri_hbm, cp_hbm, cn_hbm, dif_hbm,
                      dis_hbm, bp_out, cp_out, cn_out, dif_out, dis_out,
                      uid_v, iid_v, ka, kb, ea, eb, d1, sem0, sem1):
        wid = lax.axis_index("s") * info.num_cores + lax.axis_index("c")
        base = wid * b_per_w
        for c in range(nch):
            pltpu.sync_copy(uid_hbm.at[pl.ds(base + c * ch, ch)], uid_v.at[c])
            pltpu.sync_copy(iid_hbm.at[pl.ds(base + c * ch, ch)], iid_v.at[c])

        def run(tbl, out, bufs, sems, idxv):
            cps = [None, None]
            cps[0] = pltpu.async_copy(tbl.at[idxv.at[0]], bufs[0], sems[0])
            if nch > 1:
                cps[1] = pltpu.async_copy(tbl.at[idxv.at[1]], bufs[1], sems[1])
            for c in range(nch):
                cps[c % 2].wait()
                pltpu.sync_copy(bufs[c % 2], out.at[pl.ds(base + c * ch, ch)])
                if c + 2 < nch:
                    cps[c % 2] = pltpu.async_copy(
                        tbl.at[idxv.at[c + 2]], bufs[c % 2], sems[c % 2])

        run(pri_hbm, bp_out, [ka, kb], [sem0, sem1], uid_v)
        run(cp_hbm, cp_out, [ea, eb], [sem0, sem1], uid_v)
        run(cn_hbm, cn_out, [ea, eb], [sem0, sem1], uid_v)
        run(dif_hbm, dif_out, [ka, kb], [sem0, sem1], iid_v)
        for c in range(nch):
            pltpu.async_copy(dis_hbm.at[iid_v.at[c]], d1, sem0).wait()
            pltpu.sync_copy(d1, dis_out.at[pl.ds(base + c * ch, ch)])

    return gather_kernel(user_ids, item_ids, priori, condi_p, condi_n,
                         item_diff_w, item_disc_w)


def _tc_compute(bp_rows, cp_rows, cn_rows, dif_rows, dis_rows, item_know,
                uc_w, uc_b, ic_w, ic_b, c1_w, c1_b, c2_w, c2_b,
                interpret=False):
    B = bp_rows.shape[0]
    bt = 2048
    grid = (B // bt,)
    sb = bt // 128

    def body(bp_ref, cp_ref, cn_ref, dif_ref, dis_ref, know_ref,
             ucw_ref, ucb_ref, icw_ref, icb_ref, c1w_ref, c1b_ref,
             c2w_ref, c2b_ref, out_ref):
        a = lax.rsqrt(1.0 + jnp.exp(-cp_ref[...]))
        b = lax.rsqrt(1.0 + jnp.exp(-cn_ref[...]))
        u = (a - b).T.reshape(_N_EDGE, sb, 128)
        v = b.T.reshape(_N_EDGE, sb, 128)
        bp = (1.0 / (1.0 + jnp.exp(-bp_ref[...]))).T.reshape(_N_KNOW, sb, 128)

        cols = [None] * _N_KNOW
        cols[0] = bp[0]
        a0 = u[0] + v[0]
        b0 = v[0]
        cols[1] = (a0 * a0 - b0 * b0) * cols[0] + b0 * b0
        for k in range(2, _N_KNOW):
            f1 = u[2 * k - 3] * cols[k - 2] + v[2 * k - 3]
            f2 = u[2 * k - 2] * cols[k - 1] + v[2 * k - 2]
            cols[k] = f1 * f2
        mastery = jnp.stack(cols, axis=0).reshape(_N_KNOW, bt).T

        know = know_ref[...]
        dn = (((1,), (1,)), ((), ()))
        uf = jnp.tanh(lax.dot_general(mastery * know, ucw_ref[...], dn,
                                      preferred_element_type=jnp.float32)
                      + ucb_ref[...])
        idiff = 1.0 / (1.0 + jnp.exp(-dif_ref[...]))
        itf = jax.nn.sigmoid(lax.dot_general(idiff * know, icw_ref[...], dn,
                                             preferred_element_type=jnp.float32)
                             + icb_ref[...])
        disc = 1.0 / (1.0 + jnp.exp(-dis_ref[...]))
        iv = (uf - itf) * disc
        x1 = jax.nn.sigmoid(lax.dot_general(iv, c1w_ref[...], dn,
                                            preferred_element_type=jnp.float32)
                            + c1b_ref[...])
        x2 = jax.nn.sigmoid(jnp.sum(x1 * c2w_ref[...], axis=1, keepdims=True)
                            + c2b_ref[...])
        out_ref[...] = x2

    def bspec(w):
        return pl.BlockSpec((bt, w), lambda i: (i, 0))

    def full(arr):
        return pl.BlockSpec(arr.shape, lambda i: (0,) * arr.ndim)

    uc_b2 = uc_b.reshape(1, -1)
    ic_b2 = ic_b.reshape(1, -1)
    c1_b2 = c1_b.reshape(1, -1)
    c2_b2 = c2_b.reshape(1, -1)

    return pl.pallas_call(
        body,
        grid=grid,
        in_specs=[
            bspec(_N_KNOW), bspec(_N_EDGE), bspec(_N_EDGE), bspec(_N_KNOW),
            bspec(1), bspec(_N_KNOW),
            full(uc_w), full(uc_b2), full(ic_w), full(ic_b2),
            full(c1_w), full(c1_b2), full(c2_w), full(c2_b2),
        ],
        out_specs=bspec(1),
        out_shape=jax.ShapeDtypeStruct((B, 1), jnp.float32),
        interpret=interpret,
    )(bp_rows, cp_rows, cn_rows, dif_rows, dis_rows, item_know,
      uc_w, uc_b2, ic_w, ic_b2, c1_w, c1_b2, c2_w, c2_b2)


def kernel(user_ids, item_ids, item_know, priori, condi_p, condi_n,
           item_diff_w, item_disc_w, uc_w, uc_b, ic_w, ic_b, c1_w, c1_b,
           c2_w, c2_b):
    bp_rows, cp_rows, cn_rows, dif_rows, dis_rows = _sc_gather(
        user_ids, item_ids, priori, condi_p, condi_n, item_diff_w,
        item_disc_w)
    return _tc_compute(bp_rows, cp_rows, cn_rows, dif_rows, dis_rows,
                       item_know, uc_w, uc_b, ic_w, ic_b, c1_w, c1_b,
                       c2_w, c2_b)


# speedup vs baseline: 7.0466x; 1.5665x over previous
import functools

import jax
import jax.numpy as jnp
from jax import lax
from jax.experimental import pallas as pl
from jax.experimental.pallas import tpu as pltpu
from jax.experimental.pallas import tpu_sc as plsc

_N_KNOW = 128
_N_EDGE = 253
_N_EDGE_B = _N_EDGE - 128


def _factor_prepass(condi_p, condi_n):
    n = condi_p.shape[0]
    rows = 2000
    grid = (n // rows,)

    def body(cp_ref, cn_ref, ua_ref, ub_ref, va_ref, vb_ref):
        a = lax.rsqrt(1.0 + jnp.exp(-cp_ref[...]))
        b = lax.rsqrt(1.0 + jnp.exp(-cn_ref[...]))
        u = a - b
        ua_ref[...] = u[:, :128]
        ub_ref[:, :_N_EDGE_B] = u[:, 128:]
        va_ref[...] = b[:, :128]
        vb_ref[:, :_N_EDGE_B] = b[:, 128:]

    out128 = jax.ShapeDtypeStruct((n, 128), jnp.float32)
    return pl.pallas_call(
        body,
        grid=grid,
        in_specs=[pl.BlockSpec((rows, _N_EDGE), lambda i: (i, 0))] * 2,
        out_specs=[pl.BlockSpec((rows, 128), lambda i: (i, 0))] * 4,
        out_shape=[out128] * 4,
    )(condi_p, condi_n)


def _sc_gather(ids, tables, widths):
    B = ids.shape[0]
    info = plsc.get_sparse_core_info()
    nw = info.num_cores * info.num_subcores
    ch = 128
    b_per_w = B // nw
    nch = b_per_w // ch

    mesh = plsc.VectorSubcoreMesh(core_axis_name="c", subcore_axis_name="s")

    out_type = tuple(
        jax.ShapeDtypeStruct((B, w), jnp.float32) for w in widths)
    dwidths = sorted(set(widths))
    scratch = [pltpu.VMEM((nch, ch), jnp.int32)]
    for w in dwidths:
        scratch += [pltpu.VMEM((ch, w), jnp.float32),
                    pltpu.VMEM((ch, w), jnp.float32)]
    scratch += [pltpu.SemaphoreType.DMA, pltpu.SemaphoreType.DMA]

    @functools.partial(pl.kernel, mesh=mesh, out_type=out_type,
                       scratch_types=scratch,
                       compiler_params=pltpu.CompilerParams(
                           use_tc_tiling_on_sc=False))
    def gather_kernel(ids_hbm, *rest):
        nt = len(tables)
        tbls = rest[:nt]
        outs = rest[nt:nt * 2]
        idx_v = rest[nt * 2]
        wbufs = {w: (rest[nt * 2 + 1 + 2 * i], rest[nt * 2 + 2 + 2 * i])
                 for i, w in enumerate(dwidths)}
        sem0, sem1 = rest[nt * 2 + 1 + 2 * len(dwidths):]
        wid = lax.axis_index("s") * info.num_cores + lax.axis_index("c")
        base = wid * b_per_w
        for c in range(nch):
            pltpu.sync_copy(ids_hbm.at[pl.ds(base + c * ch, ch)], idx_v.at[c])

        for tbl, out, w in zip(tbls, outs, widths):
            bufs = list(wbufs[w])
            sems = [sem0, sem1]
            cps = [None, None]
            cps[0] = pltpu.async_copy(tbl.at[idx_v.at[0]], bufs[0], sems[0])
            if nch > 1:
                cps[1] = pltpu.async_copy(tbl.at[idx_v.at[1]], bufs[1],
                                          sems[1])
            for c in range(nch):
                cps[c % 2].wait()
                pltpu.sync_copy(bufs[c % 2], out.at[pl.ds(base + c * ch, ch)])
                if c + 2 < nch:
                    cps[c % 2] = pltpu.async_copy(
                        tbl.at[idx_v.at[c + 2]], bufs[c % 2], sems[c % 2])

    return gather_kernel(ids, *tables)


def _tc_compute(bp_rows, ua_rows, ub_rows, va_rows, vb_rows, dif_rows,
                dis_rows, item_know, uc_w, uc_b, ic_w, ic_b, c1_w, c1_b,
                c2_w, c2_b, interpret=False):
    B = bp_rows.shape[0]
    bt = 2048
    grid = (B // bt,)
    sb = bt // 128

    def body(bp_ref, ua_ref, ub_ref, va_ref, vb_ref, dif_ref, dis_ref,
             know_ref, ucw_ref, ucb_ref, icw_ref, icb_ref, c1w_ref, c1b_ref,
             c2w_ref, c2b_ref, out_ref):
        uta = ua_ref[...].T.reshape(128, sb, 128)
        utb = ub_ref[...].T.reshape(128, sb, 128)
        vta = va_ref[...].T.reshape(128, sb, 128)
        vtb = vb_ref[...].T.reshape(128, sb, 128)
        bp = (1.0 / (1.0 + jnp.exp(-bp_ref[...]))).T.reshape(_N_KNOW, sb, 128)

        def u(e):
            return uta[e] if e < 128 else utb[e - 128]

        def v(e):
            return vta[e] if e < 128 else vtb[e - 128]

        cols = [None] * _N_KNOW
        cols[0] = bp[0]
        a0 = u(0) + v(0)
        b0 = v(0)
        cols[1] = (a0 * a0 - b0 * b0) * cols[0] + b0 * b0
        for k in range(2, _N_KNOW):
            f1 = u(2 * k - 3) * cols[k - 2] + v(2 * k - 3)
            f2 = u(2 * k - 2) * cols[k - 1] + v(2 * k - 2)
            cols[k] = f1 * f2
        mastery = jnp.stack(cols, axis=0).reshape(_N_KNOW, bt).T

        know = know_ref[...]
        dn = (((1,), (1,)), ((), ()))
        uf = jnp.tanh(lax.dot_general(mastery * know, ucw_ref[...], dn,
                                      preferred_element_type=jnp.float32)
                      + ucb_ref[...])
        idiff = 1.0 / (1.0 + jnp.exp(-dif_ref[...]))
        itf = jax.nn.sigmoid(lax.dot_general(idiff * know, icw_ref[...], dn,
                                             preferred_element_type=jnp.float32)
                             + icb_ref[...])
        disc = 1.0 / (1.0 + jnp.exp(-dis_ref[...]))
        iv = (uf - itf) * disc
        x1 = jax.nn.sigmoid(lax.dot_general(iv, c1w_ref[...], dn,
                                            preferred_element_type=jnp.float32)
                            + c1b_ref[...])
        x2 = jax.nn.sigmoid(jnp.sum(x1 * c2w_ref[...], axis=1, keepdims=True)
                            + c2b_ref[...])
        out_ref[...] = x2

    def bspec(w):
        return pl.BlockSpec((bt, w), lambda i: (i, 0))

    def full(arr):
        return pl.BlockSpec(arr.shape, lambda i: (0,) * arr.ndim)

    uc_b2 = uc_b.reshape(1, -1)
    ic_b2 = ic_b.reshape(1, -1)
    c1_b2 = c1_b.reshape(1, -1)
    c2_b2 = c2_b.reshape(1, -1)

    return pl.pallas_call(
        body,
        grid=grid,
        in_specs=[
            bspec(128), bspec(128), bspec(128), bspec(128), bspec(128),
            bspec(128), bspec(1), bspec(128),
            full(uc_w), full(uc_b2), full(ic_w), full(ic_b2),
            full(c1_w), full(c1_b2), full(c2_w), full(c2_b2),
        ],
        out_specs=bspec(1),
        out_shape=jax.ShapeDtypeStruct((B, 1), jnp.float32),
        interpret=interpret,
    )(bp_rows, ua_rows, ub_rows, va_rows, vb_rows, dif_rows, dis_rows,
      item_know, uc_w, uc_b2, ic_w, ic_b2, c1_w, c1_b2, c2_w, c2_b2)


def kernel(user_ids, item_ids, item_know, priori, condi_p, condi_n,
           item_diff_w, item_disc_w, uc_w, uc_b, ic_w, ic_b, c1_w, c1_b,
           c2_w, c2_b):
    ua, ub, va, vb = _factor_prepass(condi_p, condi_n)
    dif_rows, dis_rows = _sc_gather(item_ids, (item_diff_w, item_disc_w),
                                    (_N_KNOW, 1))
    bp_rows, ua_r, ub_r, va_r, vb_r = _sc_gather(
        user_ids, (priori, ua, ub, va, vb), (128, 128, 128, 128, 128))
    return _tc_compute(bp_rows, ua_r, ub_r, va_r, vb_r, dif_rows, dis_rows,
                       item_know, uc_w, uc_b, ic_w, ic_b, c1_w, c1_b,
                       c2_w, c2_b)


# speedup vs baseline: 9.0178x; 1.2797x over previous
import functools

import jax
import jax.numpy as jnp
from jax import lax
from jax.experimental import pallas as pl
from jax.experimental.pallas import tpu as pltpu
from jax.experimental.pallas import tpu_sc as plsc

_N_KNOW = 128
_N_EDGE = 253
_N_EDGE_B = _N_EDGE - 128


def _factor_prepass(condi_p, condi_n, item_disc_w):
    n = condi_p.shape[0]
    rows = 2000
    grid = (n // rows,)

    def pack(u, v):
        ub = lax.bitcast_convert_type(u, jnp.int32) + 0x8000
        vb = lax.bitcast_convert_type(v, jnp.int32) + 0x8000
        return (ub & jnp.int32(-65536)) | ((vb >> 16) & 0xFFFF)

    def body(cp_ref, cn_ref, dis_ref, pa_ref, pb_ref, dsc_ref):
        a = lax.rsqrt(1.0 + jnp.exp(-cp_ref[...]))
        b = lax.rsqrt(1.0 + jnp.exp(-cn_ref[...]))
        u = a - b
        p = pack(u, b)
        pa_ref[...] = p[:, :128]
        pb_ref[:, :_N_EDGE_B] = p[:, 128:]
        dsc_ref[:, :1] = 1.0 / (1.0 + jnp.exp(-dis_ref[...]))

    return pl.pallas_call(
        body,
        grid=grid,
        in_specs=[pl.BlockSpec((rows, _N_EDGE), lambda i: (i, 0))] * 2
        + [pl.BlockSpec((rows, 1), lambda i: (i, 0))],
        out_specs=[pl.BlockSpec((rows, 128), lambda i: (i, 0))] * 3,
        out_shape=[jax.ShapeDtypeStruct((n, 128), jnp.int32)] * 2
        + [jax.ShapeDtypeStruct((n, 128), jnp.float32)],
    )(condi_p, condi_n, item_disc_w)


def _sc_gather(user_ids, item_ids, tables, sel):
    B = user_ids.shape[0]
    info = plsc.get_sparse_core_info()
    nw = info.num_cores * info.num_subcores
    ch = 128
    b_per_w = B // nw
    nch = b_per_w // ch

    mesh = plsc.VectorSubcoreMesh(core_axis_name="c", subcore_axis_name="s")

    kinds = [(t.shape[1], t.dtype) for t in tables]
    out_type = tuple(
        jax.ShapeDtypeStruct((B, w), dt) for w, dt in kinds)
    dkinds = sorted(set(kinds), key=str)
    scratch = [pltpu.VMEM((nch, ch), jnp.int32),
               pltpu.VMEM((nch, ch), jnp.int32)]
    for w, dt in dkinds:
        scratch += [pltpu.VMEM((ch, w), dt), pltpu.VMEM((ch, w), dt)]
    scratch += [pltpu.SemaphoreType.DMA, pltpu.SemaphoreType.DMA]

    @functools.partial(pl.kernel, mesh=mesh, out_type=out_type,
                       scratch_types=scratch,
                       compiler_params=pltpu.CompilerParams(
                           use_tc_tiling_on_sc=False))
    def gather_kernel(uid_hbm, iid_hbm, *rest):
        nt = len(tables)
        tbls = rest[:nt]
        outs = rest[nt:nt * 2]
        idx_u = rest[nt * 2]
        idx_i = rest[nt * 2 + 1]
        kbufs = {k: (rest[nt * 2 + 2 + 2 * i], rest[nt * 2 + 3 + 2 * i])
                 for i, k in enumerate(dkinds)}
        sem0, sem1 = rest[nt * 2 + 2 + 2 * len(dkinds):]
        wid = lax.axis_index("s") * info.num_cores + lax.axis_index("c")
        base = wid * b_per_w
        for c in range(nch):
            pltpu.sync_copy(uid_hbm.at[pl.ds(base + c * ch, ch)], idx_u.at[c])
            pltpu.sync_copy(iid_hbm.at[pl.ds(base + c * ch, ch)], idx_i.at[c])

        for tbl, out, k, s in zip(tbls, outs, kinds, sel):
            idx_v = idx_u if s == 0 else idx_i
            bufs = list(kbufs[k])
            sems = [sem0, sem1]
            cps = [None, None]
            cps[0] = pltpu.async_copy(tbl.at[idx_v.at[0]], bufs[0], sems[0])
            if nch > 1:
                cps[1] = pltpu.async_copy(tbl.at[idx_v.at[1]], bufs[1],
                                          sems[1])
            for c in range(nch):
                cps[c % 2].wait()
                pltpu.sync_copy(bufs[c % 2], out.at[pl.ds(base + c * ch, ch)])
                if c + 2 < nch:
                    cps[c % 2] = pltpu.async_copy(
                        tbl.at[idx_v.at[c + 2]], bufs[c % 2], sems[c % 2])

    return gather_kernel(user_ids, item_ids, *tables)


def _tc_compute(bp_rows, pa_rows, pb_rows, dif_rows, dis_rows, item_know,
                uc_w, uc_b, ic_w, ic_b, c1_w, c1_b, c2_w, c2_b,
                interpret=False):
    B = bp_rows.shape[0]
    bt = 2048
    grid = (B // bt,)
    sb = bt // 128

    def body(bp_ref, pa_ref, pb_ref, dif_ref, dis_ref,
             know_ref, ucw_ref, ucb_ref, icw_ref, icb_ref, c1w_ref, c1b_ref,
             c2w_ref, c2b_ref, out_ref):
        pta = pa_ref[...].T.reshape(128, sb, 128)
        ptb = pb_ref[...].T.reshape(128, sb, 128)
        bp = (1.0 / (1.0 + jnp.exp(-bp_ref[...]))).T.reshape(_N_KNOW, sb, 128)

        def word(e):
            return pta[e] if e < 128 else ptb[e - 128]

        def u(e):
            return lax.bitcast_convert_type(word(e) & jnp.int32(-65536),
                                            jnp.float32)

        def v(e):
            return lax.bitcast_convert_type(word(e) << 16, jnp.float32)

        cols = [None] * _N_KNOW
        cols[0] = bp[0]
        a0 = u(0) + v(0)
        b0 = v(0)
        cols[1] = (a0 * a0 - b0 * b0) * cols[0] + b0 * b0
        for k in range(2, _N_KNOW):
            f1 = u(2 * k - 3) * cols[k - 2] + v(2 * k - 3)
            f2 = u(2 * k - 2) * cols[k - 1] + v(2 * k - 2)
            cols[k] = f1 * f2
        mastery = jnp.stack(cols, axis=0).reshape(_N_KNOW, bt).T

        know = know_ref[...]
        dn = (((1,), (1,)), ((), ()))
        uf = jnp.tanh(lax.dot_general(mastery * know, ucw_ref[...], dn,
                                      preferred_element_type=jnp.float32)
                      + ucb_ref[...])
        idiff = 1.0 / (1.0 + jnp.exp(-dif_ref[...]))
        itf = jax.nn.sigmoid(lax.dot_general(idiff * know, icw_ref[...], dn,
                                             preferred_element_type=jnp.float32)
                             + icb_ref[...])
        disc = dis_ref[:, :1]
        iv = (uf - itf) * disc
        x1 = jax.nn.sigmoid(lax.dot_general(iv, c1w_ref[...], dn,
                                            preferred_element_type=jnp.float32)
                            + c1b_ref[...])
        x2 = jax.nn.sigmoid(jnp.sum(x1 * c2w_ref[...], axis=1, keepdims=True)
                            + c2b_ref[...])
        out_ref[...] = x2

    def bspec(w):
        return pl.BlockSpec((bt, w), lambda i: (i, 0))

    def full(arr):
        return pl.BlockSpec(arr.shape, lambda i: (0,) * arr.ndim)

    uc_b2 = uc_b.reshape(1, -1)
    ic_b2 = ic_b.reshape(1, -1)
    c1_b2 = c1_b.reshape(1, -1)
    c2_b2 = c2_b.reshape(1, -1)

    return pl.pallas_call(
        body,
        grid=grid,
        in_specs=[
            bspec(128), bspec(128), bspec(128), bspec(128), bspec(128),
            bspec(128),
            full(uc_w), full(uc_b2), full(ic_w), full(ic_b2),
            full(c1_w), full(c1_b2), full(c2_w), full(c2_b2),
        ],
        out_specs=bspec(1),
        out_shape=jax.ShapeDtypeStruct((B, 1), jnp.float32),
        interpret=interpret,
    )(bp_rows, pa_rows, pb_rows, dif_rows, dis_rows,
      item_know, uc_w, uc_b2, ic_w, ic_b2, c1_w, c1_b2, c2_w, c2_b2)


def kernel(user_ids, item_ids, item_know, priori, condi_p, condi_n,
           item_diff_w, item_disc_w, uc_w, uc_b, ic_w, ic_b, c1_w, c1_b,
           c2_w, c2_b):
    pa, pb, dsc = _factor_prepass(condi_p, condi_n, item_disc_w)
    bp_rows, dif_rows = _sc_gather(user_ids, item_ids,
                                   (priori, item_diff_w), (0, 1))
    pa_r, pb_r, dis_rows = _sc_gather(user_ids, item_ids,
                                      (pa, pb, dsc), (0, 0, 1))
    return _tc_compute(bp_rows, pa_r, pb_r, dif_rows, dis_rows,
                       item_know, uc_w, uc_b, ic_w, ic_b, c1_w, c1_b,
                       c2_w, c2_b)


# speedup vs baseline: 10.2430x; 1.1359x over previous
import functools

import jax
import jax.numpy as jnp
from jax import lax
from jax.experimental import pallas as pl
from jax.experimental.pallas import tpu as pltpu
from jax.experimental.pallas import tpu_sc as plsc

_N_KNOW = 128
_N_EDGE = 253
_N_EDGE_B = _N_EDGE - 128


def _factor_prepass(condi_p, condi_n, item_disc_w):
    n = condi_p.shape[0]
    rows = 2000
    grid = (n // rows,)

    def pack(u, v):
        ub = lax.bitcast_convert_type(u, jnp.int32) + 0x8000
        vb = lax.bitcast_convert_type(v, jnp.int32) + 0x8000
        return (ub & jnp.int32(-65536)) | ((vb >> 16) & 0xFFFF)

    def body(cp_ref, cn_ref, dis_ref, pa_ref, pb_ref, dsc_ref):
        a = lax.rsqrt(1.0 + jnp.exp(-cp_ref[...]))
        b = lax.rsqrt(1.0 + jnp.exp(-cn_ref[...]))
        u = a - b
        p = pack(u, b)
        pa_ref[...] = p[:, :128]
        pb_ref[:, :_N_EDGE_B] = p[:, 128:]
        dis = dis_ref[0, 0, :]
        dsc_ref[:, :1] = (1.0 / (1.0 + jnp.exp(-dis)))[:, None]

    return pl.pallas_call(
        body,
        grid=grid,
        in_specs=[pl.BlockSpec((rows, _N_EDGE), lambda i: (i, 0))] * 2
        + [pl.BlockSpec((1, 1, rows), lambda i: (i, 0, 0))],
        out_specs=[pl.BlockSpec((rows, 128), lambda i: (i, 0))] * 3,
        out_shape=[jax.ShapeDtypeStruct((n, 128), jnp.int32)] * 2
        + [jax.ShapeDtypeStruct((n, 128), jnp.float32)],
    )(condi_p, condi_n, item_disc_w.reshape(n // rows, 1, rows))


def _sc_gather(user_ids, item_ids, tables, sel):
    B = user_ids.shape[0]
    info = plsc.get_sparse_core_info()
    nw = info.num_cores * info.num_subcores
    ch = 128
    b_per_w = B // nw
    nch = b_per_w // ch

    mesh = plsc.VectorSubcoreMesh(core_axis_name="c", subcore_axis_name="s")

    kinds = [(t.shape[1], t.dtype) for t in tables]
    out_type = tuple(
        jax.ShapeDtypeStruct((B, w), dt) for w, dt in kinds)
    dkinds = sorted(set(kinds), key=str)
    scratch = [pltpu.VMEM((nch, ch), jnp.int32),
               pltpu.VMEM((nch, ch), jnp.int32)]
    for w, dt in dkinds:
        scratch += [pltpu.VMEM((ch, w), dt), pltpu.VMEM((ch, w), dt)]
    scratch += [pltpu.SemaphoreType.DMA, pltpu.SemaphoreType.DMA]

    @functools.partial(pl.kernel, mesh=mesh, out_type=out_type,
                       scratch_types=scratch,
                       compiler_params=pltpu.CompilerParams(
                           use_tc_tiling_on_sc=False))
    def gather_kernel(uid_hbm, iid_hbm, *rest):
        nt = len(tables)
        tbls = rest[:nt]
        outs = rest[nt:nt * 2]
        idx_u = rest[nt * 2]
        idx_i = rest[nt * 2 + 1]
        kbufs = {k: (rest[nt * 2 + 2 + 2 * i], rest[nt * 2 + 3 + 2 * i])
                 for i, k in enumerate(dkinds)}
        sem0, sem1 = rest[nt * 2 + 2 + 2 * len(dkinds):]
        wid = lax.axis_index("s") * info.num_cores + lax.axis_index("c")
        base = wid * b_per_w
        for c in range(nch):
            pltpu.sync_copy(uid_hbm.at[pl.ds(base + c * ch, ch)], idx_u.at[c])
            pltpu.sync_copy(iid_hbm.at[pl.ds(base + c * ch, ch)], idx_i.at[c])

        for tbl, out, k, s in zip(tbls, outs, kinds, sel):
            idx_v = idx_u if s == 0 else idx_i
            bufs = list(kbufs[k])
            sems = [sem0, sem1]
            cps = [None, None]
            cps[0] = pltpu.async_copy(tbl.at[idx_v.at[0]], bufs[0], sems[0])
            if nch > 1:
                cps[1] = pltpu.async_copy(tbl.at[idx_v.at[1]], bufs[1],
                                          sems[1])
            for c in range(nch):
                cps[c % 2].wait()
                pltpu.sync_copy(bufs[c % 2], out.at[pl.ds(base + c * ch, ch)])
                if c + 2 < nch:
                    cps[c % 2] = pltpu.async_copy(
                        tbl.at[idx_v.at[c + 2]], bufs[c % 2], sems[c % 2])

    return gather_kernel(user_ids, item_ids, *tables)


def _tc_compute(bp_rows, pa_rows, pb_rows, dif_rows, dis_rows, item_know,
                uc_w, uc_b, ic_w, ic_b, c1_w, c1_b, c2_w, c2_b,
                interpret=False):
    B = bp_rows.shape[0]
    bt = 2048
    grid = (B // bt,)
    sb = bt // 128

    def body(bp_ref, pa_ref, pb_ref, dif_ref, dis_ref,
             know_ref, ucw_ref, ucb_ref, icw_ref, icb_ref, c1w_ref, c1b_ref,
             c2w_ref, c2b_ref, out_ref):
        pta = pa_ref[...].T.reshape(128, sb, 128)
        ptb = pb_ref[...].T.reshape(128, sb, 128)
        bp = (1.0 / (1.0 + jnp.exp(-bp_ref[...]))).T.reshape(_N_KNOW, sb, 128)

        def word(e):
            return pta[e] if e < 128 else ptb[e - 128]

        def u(e):
            return lax.bitcast_convert_type(word(e) & jnp.int32(-65536),
                                            jnp.float32)

        def v(e):
            return lax.bitcast_convert_type(word(e) << 16, jnp.float32)

        cols = [None] * _N_KNOW
        cols[0] = bp[0]
        a0 = u(0) + v(0)
        b0 = v(0)
        cols[1] = (a0 * a0 - b0 * b0) * cols[0] + b0 * b0
        for k in range(2, _N_KNOW):
            f1 = u(2 * k - 3) * cols[k - 2] + v(2 * k - 3)
            f2 = u(2 * k - 2) * cols[k - 1] + v(2 * k - 2)
            cols[k] = f1 * f2
        mastery = jnp.stack(cols, axis=0).reshape(_N_KNOW, bt).T

        know = know_ref[...]
        dn = (((1,), (1,)), ((), ()))
        uf = jnp.tanh(lax.dot_general(mastery * know, ucw_ref[...], dn,
                                      preferred_element_type=jnp.float32)
                      + ucb_ref[...])
        idiff = 1.0 / (1.0 + jnp.exp(-dif_ref[...]))
        itf = jax.nn.sigmoid(lax.dot_general(idiff * know, icw_ref[...], dn,
                                             preferred_element_type=jnp.float32)
                             + icb_ref[...])
        disc = dis_ref[:, :1]
        iv = (uf - itf) * disc
        x1 = jax.nn.sigmoid(lax.dot_general(iv, c1w_ref[...], dn,
                                            preferred_element_type=jnp.float32)
                            + c1b_ref[...])
        x2 = jax.nn.sigmoid(jnp.sum(x1 * c2w_ref[...], axis=1, keepdims=True)
                            + c2b_ref[...])
        out_ref[...] = x2

    def bspec(w):
        return pl.BlockSpec((bt, w), lambda i: (i, 0))

    def full(arr):
        return pl.BlockSpec(arr.shape, lambda i: (0,) * arr.ndim)

    uc_b2 = uc_b.reshape(1, -1)
    ic_b2 = ic_b.reshape(1, -1)
    c1_b2 = c1_b.reshape(1, -1)
    c2_b2 = c2_b.reshape(1, -1)

    return pl.pallas_call(
        body,
        grid=grid,
        in_specs=[
            bspec(128), bspec(128), bspec(128), bspec(128), bspec(128),
            bspec(128),
            full(uc_w), full(uc_b2), full(ic_w), full(ic_b2),
            full(c1_w), full(c1_b2), full(c2_w), full(c2_b2),
        ],
        out_specs=bspec(1),
        out_shape=jax.ShapeDtypeStruct((B, 1), jnp.float32),
        interpret=interpret,
    )(bp_rows, pa_rows, pb_rows, dif_rows, dis_rows,
      item_know, uc_w, uc_b2, ic_w, ic_b2, c1_w, c1_b2, c2_w, c2_b2)


def kernel(user_ids, item_ids, item_know, priori, condi_p, condi_n,
           item_diff_w, item_disc_w, uc_w, uc_b, ic_w, ic_b, c1_w, c1_b,
           c2_w, c2_b):
    pa, pb, dsc = _factor_prepass(condi_p, condi_n, item_disc_w)
    bp_rows, dif_rows, pa_r, pb_r, dis_rows = _sc_gather(
        user_ids, item_ids, (priori, item_diff_w, pa, pb, dsc),
        (0, 1, 0, 0, 1))
    return _tc_compute(bp_rows, pa_r, pb_r, dif_rows, dis_rows,
                       item_know, uc_w, uc_b, ic_w, ic_b, c1_w, c1_b,
                       c2_w, c2_b)
